# baseline (speedup 1.0000x reference)
import functools

import jax
import jax.numpy as jnp
from jax.experimental import pallas as pl
from jax.experimental.pallas import tpu as pltpu

TN = 2048


def _body(x_ref, dp_ref, cb_ref, dm_ref, w1_ref, b1_ref, w2_ref, b2_ref,
          o_ref, g0_scr, g1_scr, *, num_experts, f):
    s = pl.program_id(1)

    @pl.when(s < num_experts)
    def _first_matmul():
        onehot = (jax.lax.broadcasted_iota(jnp.int32, (1, num_experts), 1)
                  == s).astype(jnp.float32)
        disp = jnp.sum(dp_ref[:] * onehot, axis=1, keepdims=True)
        comb = jnp.sum(cb_ref[:] * onehot, axis=1, keepdims=True)
        dmask = jnp.sum(dm_ref[:] * onehot, axis=1, keepdims=True)

        h0 = jnp.dot(x_ref[...], w1_ref[0].astype(jnp.bfloat16),
                     preferred_element_type=jnp.float32)

        mask = (dmask != 0.0).astype(jnp.float32)

        h = h0 * disp + mask * b1_ref[0, 0][None, :]
        g = (0.5 * comb) * h * (1.0 + jax.lax.erf(h * 0.7071067811865476))
        gb = g.astype(jnp.bfloat16)

        @pl.when(s % 2 == 0)
        def _even():
            g0_scr[...] = gb

        @pl.when(s % 2 == 1)
        def _odd():
            g1_scr[...] = gb

    @pl.when(s > 0)
    def _second_matmul():
        w2b = w2_ref[0].astype(jnp.bfloat16)

        def dot_from(scr):
            return jnp.dot(scr[...], w2b,
                           preferred_element_type=jnp.float32)

        @pl.when(s % 2 == 1)
        def _from_even():
            y = dot_from(g0_scr)

            @pl.when(s == 1)
            def _init():
                o_ref[...] = y + b2_ref[0][None, :]

            @pl.when(s > 1)
            def _acc():
                o_ref[...] += y

        @pl.when(s % 2 == 0)
        def _from_odd():
            o_ref[...] += dot_from(g1_scr)


@jax.jit
def kernel(x, dispatch_tensor, combine_tensor, w1, b1, w2, b2):
    b, n, h = x.shape
    e, _, f = w1.shape
    tn = TN
    num_t = n // tn

    x2 = x.reshape(n, h)
    xb = x2.astype(jnp.bfloat16)
    dp = dispatch_tensor.reshape(n, e)
    cb = combine_tensor.reshape(n, e)
    dm = dp * jnp.sum(x2, axis=-1, keepdims=True)
    b1r = b1.reshape(e, 1, f)
    b2r = b2.reshape(1, h)

    def w1map(ti, si):
        return (jnp.minimum(si, e - 1), 0, 0)

    def w2map(ti, si):
        return (jnp.maximum(si, 1) - 1, 0, 0)

    out = pl.pallas_call(
        functools.partial(_body, num_experts=e, f=f),
        grid=(num_t, e + 1),
        in_specs=[
            pl.BlockSpec((tn, h), lambda ti, si: (ti, 0)),
            pl.BlockSpec((tn, e), lambda ti, si: (ti, 0)),
            pl.BlockSpec((tn, e), lambda ti, si: (ti, 0)),
            pl.BlockSpec((tn, e), lambda ti, si: (ti, 0)),
            pl.BlockSpec((1, h, f), w1map),
            pl.BlockSpec((1, 1, f), w1map),
            pl.BlockSpec((1, f, h), w2map),
            pl.BlockSpec((1, h), lambda ti, si: (0, 0)),
        ],
        out_specs=pl.BlockSpec((tn, h), lambda ti, si: (ti, 0)),
        out_shape=jax.ShapeDtypeStruct((n, h), jnp.float32),
        scratch_shapes=[
            pltpu.VMEM((tn, f), jnp.bfloat16),
            pltpu.VMEM((tn, f), jnp.bfloat16),
        ],
        compiler_params=pltpu.CompilerParams(
            dimension_semantics=("arbitrary", "arbitrary"),
        ),
    )(xb, dp, cb, dm, w1, b1r, w2, b2r)

    return out.reshape(b, n, h)
